# baseline (speedup 1.0000x reference)
import functools

import jax
import jax.numpy as jnp
from jax import lax
from jax.experimental import pallas as pl
from jax.experimental.pallas import tpu as pltpu
from jax.experimental.pallas import tpu_sc as plsc

BATCH = 16384
D = 64
L = 16
NC = 2
NS = 16
NW = NC * NS
BPW = BATCH // NW
PASS_ROWS = 256
NPASS = BPW // PASS_ROWS
NGP = PASS_ROWS // L


def _nr_rsqrt(x):
    i = lax.bitcast_convert_type(x, jnp.int32)
    i = jnp.int32(0x5F3759DF) - lax.shift_right_logical(i, 1)
    y = lax.bitcast_convert_type(i, jnp.float32)
    for _ in range(3):
        y = y * (jnp.float32(1.5) - jnp.float32(0.5) * x * y * y)
    return y


def _sc_body(uids_hbm, rids_hbm, utab_hbm, rtab_hbm, out_hbm,
             uidx_v, ridx_v, urows_v, rrows_v, out_v, sem):
    wid = lax.axis_index("s") * NC + lax.axis_index("c")
    base = wid * BPW

    pltpu.sync_copy(uids_hbm.at[pl.ds(base, BPW)], uidx_v)
    pltpu.sync_copy(rids_hbm.at[pl.ds(base, BPW)], ridx_v)

    lanes = lax.iota(jnp.int32, L)
    eps2 = jnp.float32(1e-16)

    for p in range(NPASS):
        def fire(t, carry):
            uvec = uidx_v[pl.ds(p * PASS_ROWS + t * L, L)]
            rvec = ridx_v[pl.ds(p * PASS_ROWS + t * L, L)]
            for i in range(L):
                slot = t * L + i
                pltpu.async_copy(utab_hbm.at[uvec[i]], urows_v.at[slot], sem)
                pltpu.async_copy(rtab_hbm.at[rvec[i]], rrows_v.at[slot], sem)
            return carry

        with jax.named_scope(f"fire{p}"):
            lax.fori_loop(0, NGP, fire, 0)

        def drain(t, carry):
            for _ in range(2 * L):
                pltpu.make_async_copy(utab_hbm.at[0], urows_v.at[0], sem).wait()
            return carry

        with jax.named_scope(f"drain{p}"):
            lax.fori_loop(0, NGP, drain, 0)

        def group(g, carry):
            row = g * L + lanes
            dot = jnp.zeros((L,), jnp.float32)
            u2 = jnp.zeros((L,), jnp.float32)
            r2 = jnp.zeros((L,), jnp.float32)
            for d in range(D):
                col = jnp.full((L,), d, jnp.int32)
                uc = plsc.load_gather(urows_v, [row, col])
                rc = plsc.load_gather(rrows_v, [row, col])
                dot = dot + uc * rc
                u2 = u2 + uc * uc
                r2 = r2 + rc * rc
            sim = (dot * _nr_rsqrt(jnp.maximum(u2, eps2))
                   * _nr_rsqrt(jnp.maximum(r2, eps2)))
            out_v[pl.ds(p * PASS_ROWS + g * L, L)] = sim
            return carry

        with jax.named_scope(f"compute{p}"):
            lax.fori_loop(0, NGP, group, 0)

    pltpu.sync_copy(out_v, out_hbm.at[pl.ds(base, BPW)])


def kernel(user_ids, reel_ids, user_table, reel_table):
    uids = user_ids.astype(jnp.int32)
    rids = reel_ids.astype(jnp.int32)
    mesh = plsc.VectorSubcoreMesh(core_axis_name="c", subcore_axis_name="s")
    fn = functools.partial(
        pl.kernel,
        mesh=mesh,
        compiler_params=pltpu.CompilerParams(
            needs_layout_passes=False, use_tc_tiling_on_sc=True),
        out_type=jax.ShapeDtypeStruct((BATCH,), jnp.float32),
        scratch_types=[
            pltpu.VMEM((BPW,), jnp.int32),
            pltpu.VMEM((BPW,), jnp.int32),
            pltpu.VMEM((PASS_ROWS, D), jnp.float32),
            pltpu.VMEM((PASS_ROWS, D), jnp.float32),
            pltpu.VMEM((BPW,), jnp.float32),
            pltpu.SemaphoreType.DMA,
        ],
    )(_sc_body)
    return fn(uids, rids, user_table, reel_table)


# speedup vs baseline: 2.0469x; 2.0469x over previous
import functools

import jax
import jax.numpy as jnp
from jax import lax
from jax.experimental import pallas as pl
from jax.experimental.pallas import tpu as pltpu
from jax.experimental.pallas import tpu_sc as plsc

BATCH = 16384
D = 64
L = 16
NC = 2
NS = 16
NW = NC * NS
BPW = BATCH // NW
NROWS = 1000001
PW = 256
PSHIFT = 8
NPAN = NROWS // PW
SLOTS = 123
MCAP = 1040
SCAN_G = BATCH // L
RING = 32


def _nr_rsqrt(x):
    i = lax.bitcast_convert_type(x, jnp.int32)
    i = jnp.int32(0x5F3759DF) - lax.shift_right_logical(i, 1)
    y = lax.bitcast_convert_type(i, jnp.float32)
    for _ in range(3):
        y = y * (jnp.float32(1.5) - jnp.float32(0.5) * x * y * y)
    return y


def _fire_panel(tab_hbm, panel_v, buf, p, sem):
    @pl.when(p < NPAN)
    def _():
        for br in range(8):
            pltpu.async_copy(
                tab_hbm.at[pl.ds(br * 8, 8), pl.ds(p * PW, PW)],
                panel_v.at[buf, pl.ds(br * 8, 8), pl.ds(0, PW)], sem)

    @pl.when(p == NPAN)
    def _():
        for d in range(D):
            pltpu.async_copy(
                tab_hbm.at[pl.ds(d, 1), pl.ds(NPAN * PW, 64)],
                panel_v.at[buf, pl.ds(d, 1), pl.ds(0, 64)], sem)


def _drain_panel(tab_hbm, panel_v, p, sem):
    @pl.when(p < NPAN)
    def _():
        for _i in range(8):
            pltpu.make_async_copy(
                tab_hbm.at[pl.ds(0, 8), pl.ds(0, PW)],
                panel_v.at[0, pl.ds(0, 8), pl.ds(0, PW)], sem).wait()

    @pl.when(p == NPAN)
    def _():
        for _i in range(D):
            pltpu.make_async_copy(
                tab_hbm.at[pl.ds(0, 1), pl.ds(0, 64)],
                panel_v.at[0, pl.ds(0, 1), pl.ds(0, 64)], sem).wait()


def _gather_body(uids_hbm, rids_hbm, utabT_hbm, rtabT_hbm, ug_hbm, rg_hbm,
                 aids_v, mr_u, mb_u, mr_r, mb_r, panel_u, panel_r,
                 pr_v, pb_v, temp_v, sem_p, sem_o):
    w = lax.axis_index("s") * NC + lax.axis_index("c")
    lanes = lax.iota(jnp.int32, L)

    def scan_ids(ids_hbm, mr_v, mb_v):
        pltpu.sync_copy(ids_hbm, aids_v)

        def scan(j, cnt):
            v = aids_v[pl.ds(j * L, L)]
            m = lax.eq(
                lax.bitwise_and(lax.shift_right_logical(v, PSHIFT), 31), w)
            plsc.store_compressed(mr_v.at[pl.ds(cnt, L)], v, mask=m)
            plsc.store_compressed(
                mb_v.at[pl.ds(cnt, L)], j * L + lanes, mask=m)
            return cnt + plsc.all_reduce_population_count(m)[0]

        return lax.fori_loop(0, SCAN_G, scan, 0)

    cnt_u = scan_ids(uids_hbm, mr_u, mb_u)
    cnt_r = scan_ids(rids_hbm, mr_r, mb_r)

    def extract(p, panel_v, buf, mr_v, mb_v, cnt, out_hbm, issued):
        def pscan(j, c2):
            valid = (j * L + lanes) < cnt
            rv = mr_v[pl.ds(j * L, L)]
            bv = mb_v[pl.ds(j * L, L)]
            m = valid & lax.eq(lax.shift_right_logical(rv, PSHIFT), p)
            plsc.store_compressed(pr_v.at[pl.ds(c2, L)], rv, mask=m)
            plsc.store_compressed(pb_v.at[pl.ds(c2, L)], bv, mask=m)
            return c2 + plsc.all_reduce_population_count(m)[0]

        cp = lax.fori_loop(0, (MCAP // L) + 1, pscan, 0)

        def per_id(i, iss):
            @pl.when(iss >= RING)
            def _():
                pltpu.make_async_copy(
                    ug_hbm.at[pl.ds(0, D)],
                    temp_v.at[pl.ds(0, D)], sem_o).wait()

            rr = pr_v[pl.ds(i, L)][0]
            bb = pb_v[pl.ds(i, L)][0]
            cl = lax.bitwise_and(rr, PW - 1)
            slot = lax.bitwise_and(iss, RING - 1)
            cs = jnp.full((L,), 0, jnp.int32) + cl
            for q in range(D // L):
                dv = q * L + lanes
                col = plsc.load_gather(panel_v.at[buf], [dv, cs])
                temp_v[pl.ds(slot * D + q * L, L)] = col
            pltpu.async_copy(
                temp_v.at[pl.ds(slot * D, D)],
                out_hbm.at[pl.ds(bb * D, D)], sem_o)
            return iss + 1

        return lax.fori_loop(0, cp, per_id, issued)

    _fire_panel(utabT_hbm, panel_u, 0, w, sem_p)
    _fire_panel(rtabT_hbm, panel_r, 0, w, sem_p)

    def sweep(s, issued):
        buf = lax.bitwise_and(s, 1)
        p = s * NW + w
        _drain_panel(utabT_hbm, panel_u, p, sem_p)
        _drain_panel(rtabT_hbm, panel_r, p, sem_p)
        p_next = (s + 1) * NW + w

        @pl.when(p_next <= NPAN)
        def _():
            nbuf = lax.bitwise_and(s + 1, 1)
            _fire_panel(utabT_hbm, panel_u, nbuf, p_next, sem_p)
            _fire_panel(rtabT_hbm, panel_r, nbuf, p_next, sem_p)

        issued = extract(p, panel_u, buf, mr_u, mb_u, cnt_u, ug_hbm, issued)
        issued = extract(p, panel_r, buf, mr_r, mb_r, cnt_r, rg_hbm, issued)
        return issued

    total = lax.fori_loop(0, SLOTS, sweep, 0)

    def fin(i, carry):
        pltpu.make_async_copy(
            ug_hbm.at[pl.ds(0, D)], temp_v.at[pl.ds(0, D)], sem_o).wait()
        return carry

    lax.fori_loop(0, jnp.minimum(total, RING), fin, 0)


def _cosine_body(ug_hbm, rg_hbm, out_hbm, urows_v, rrows_v, out_v):
    w = lax.axis_index("s") * NC + lax.axis_index("c")
    base = w * BPW
    pltpu.sync_copy(ug_hbm.at[pl.ds(base * D, BPW * D)], urows_v)
    pltpu.sync_copy(rg_hbm.at[pl.ds(base * D, BPW * D)], rrows_v)

    lanes = lax.iota(jnp.int32, L)
    eps2 = jnp.float32(1e-16)

    def group(g, carry):
        flat = (g * L + lanes) * D
        dot = jnp.zeros((L,), jnp.float32)
        u2 = jnp.zeros((L,), jnp.float32)
        r2 = jnp.zeros((L,), jnp.float32)
        for d in range(D):
            uc = plsc.load_gather(urows_v, [flat + d])
            rc = plsc.load_gather(rrows_v, [flat + d])
            dot = dot + uc * rc
            u2 = u2 + uc * uc
            r2 = r2 + rc * rc
        sim = (dot * _nr_rsqrt(jnp.maximum(u2, eps2))
               * _nr_rsqrt(jnp.maximum(r2, eps2)))
        out_v[pl.ds(g * L, L)] = sim
        return carry

    lax.fori_loop(0, BPW // L, group, 0)
    pltpu.sync_copy(out_v, out_hbm.at[pl.ds(base, BPW)])


def kernel(user_ids, reel_ids, user_table, reel_table):
    uids = user_ids.astype(jnp.int32)
    rids = reel_ids.astype(jnp.int32)
    mesh = plsc.VectorSubcoreMesh(core_axis_name="c", subcore_axis_name="s")
    params = pltpu.CompilerParams(
        needs_layout_passes=False, use_tc_tiling_on_sc=True)

    gather_fn = functools.partial(
        pl.kernel,
        mesh=mesh,
        compiler_params=params,
        out_type=(
            jax.ShapeDtypeStruct((BATCH * D,), jnp.float32),
            jax.ShapeDtypeStruct((BATCH * D,), jnp.float32),
        ),
        scratch_types=[
            pltpu.VMEM((BATCH,), jnp.int32),
            pltpu.VMEM((MCAP + L,), jnp.int32),
            pltpu.VMEM((MCAP + L,), jnp.int32),
            pltpu.VMEM((MCAP + L,), jnp.int32),
            pltpu.VMEM((MCAP + L,), jnp.int32),
            pltpu.VMEM((2, D, PW), jnp.float32),
            pltpu.VMEM((2, D, PW), jnp.float32),
            pltpu.VMEM((MCAP + L,), jnp.int32),
            pltpu.VMEM((MCAP + L,), jnp.int32),
            pltpu.VMEM((RING * D,), jnp.float32),
            pltpu.SemaphoreType.DMA,
            pltpu.SemaphoreType.DMA,
        ],
    )(_gather_body)
    ug, rg = gather_fn(uids, rids, user_table.T, reel_table.T)

    cos_fn = functools.partial(
        pl.kernel,
        mesh=mesh,
        compiler_params=params,
        out_type=jax.ShapeDtypeStruct((BATCH,), jnp.float32),
        scratch_types=[
            pltpu.VMEM((BPW * D,), jnp.float32),
            pltpu.VMEM((BPW * D,), jnp.float32),
            pltpu.VMEM((BPW,), jnp.float32),
        ],
    )(_cosine_body)
    return cos_fn(ug, rg)


# speedup vs baseline: 2.1589x; 1.0547x over previous
import functools

import jax
import jax.numpy as jnp
from jax import lax
from jax.experimental import pallas as pl
from jax.experimental.pallas import tpu as pltpu
from jax.experimental.pallas import tpu_sc as plsc

BATCH = 16384
D = 64
L = 16
NC = 2
NS = 16
NW = NC * NS
BPW = BATCH // NW
NROWS = 1000001
PW = 256
PSHIFT = 8
NPAN = NROWS // PW
SLOTS = 123
MCAP = 1040
SCAN_G = BATCH // L
RING = 32


def _nr_rsqrt(x):
    i = lax.bitcast_convert_type(x, jnp.int32)
    i = jnp.int32(0x5F3759DF) - lax.shift_right_logical(i, 1)
    y = lax.bitcast_convert_type(i, jnp.float32)
    for _ in range(3):
        y = y * (jnp.float32(1.5) - jnp.float32(0.5) * x * y * y)
    return y


def _fire_panel(tab_hbm, panel_v, buf, p, sem):
    @pl.when(p < NPAN)
    def _():
        for br in range(8):
            pltpu.async_copy(
                tab_hbm.at[pl.ds(br * 8, 8), pl.ds(p * PW, PW)],
                panel_v.at[buf, pl.ds(br * 8, 8), pl.ds(0, PW)], sem)

    @pl.when(p == NPAN)
    def _():
        for d in range(D):
            pltpu.async_copy(
                tab_hbm.at[pl.ds(d, 1), pl.ds(NPAN * PW, 64)],
                panel_v.at[buf, pl.ds(d, 1), pl.ds(0, 64)], sem)


def _drain_panel(tab_hbm, panel_v, p, sem):
    @pl.when(p < NPAN)
    def _():
        for _i in range(8):
            pltpu.make_async_copy(
                tab_hbm.at[pl.ds(0, 8), pl.ds(0, PW)],
                panel_v.at[0, pl.ds(0, 8), pl.ds(0, PW)], sem).wait()

    @pl.when(p == NPAN)
    def _():
        for _i in range(D):
            pltpu.make_async_copy(
                tab_hbm.at[pl.ds(0, 1), pl.ds(0, 64)],
                panel_v.at[0, pl.ds(0, 1), pl.ds(0, 64)], sem).wait()


def _gather_body(uids_hbm, rids_hbm, utabT_hbm, rtabT_hbm, ug_hbm, rg_hbm,
                 aids_v, mr_u, mb_u, mr_r, mb_r, panel_u, panel_r,
                 pr_v, pb_v, temp_v, sem_p, sem_o):
    w = lax.axis_index("s") * NC + lax.axis_index("c")
    lanes = lax.iota(jnp.int32, L)

    _fire_panel(utabT_hbm, panel_u, 0, w, sem_p)
    _fire_panel(rtabT_hbm, panel_r, 0, w, sem_p)

    def scan_ids(ids_hbm, mr_v, mb_v):
        pltpu.sync_copy(ids_hbm, aids_v)

        def scan(j, cnt):
            v = aids_v[pl.ds(j * L, L)]
            m = lax.eq(
                lax.bitwise_and(lax.shift_right_logical(v, PSHIFT), 31), w)
            plsc.store_compressed(mr_v.at[pl.ds(cnt, L)], v, mask=m)
            plsc.store_compressed(
                mb_v.at[pl.ds(cnt, L)], j * L + lanes, mask=m)
            return cnt + plsc.all_reduce_population_count(m)[0]

        return lax.fori_loop(0, SCAN_G, scan, 0)

    cnt_u = scan_ids(uids_hbm, mr_u, mb_u)
    cnt_r = scan_ids(rids_hbm, mr_r, mb_r)

    def extract(p, panel_v, buf, mr_v, mb_v, cnt, out_hbm, issued):
        def pscan(j, c2):
            valid = (j * L + lanes) < cnt
            rv = mr_v[pl.ds(j * L, L)]
            bv = mb_v[pl.ds(j * L, L)]
            m = valid & lax.eq(lax.shift_right_logical(rv, PSHIFT), p)
            plsc.store_compressed(pr_v.at[pl.ds(c2, L)], rv, mask=m)
            plsc.store_compressed(pb_v.at[pl.ds(c2, L)], bv, mask=m)
            return c2 + plsc.all_reduce_population_count(m)[0]

        cp = lax.fori_loop(0, lax.shift_right_logical(cnt + (L - 1), 4),
                           pscan, 0)

        def per_id(i, iss):
            @pl.when(iss >= RING)
            def _():
                pltpu.make_async_copy(
                    ug_hbm.at[pl.ds(0, D)],
                    temp_v.at[pl.ds(0, D)], sem_o).wait()

            rr = pr_v[pl.ds(i, L)][0]
            bb = pb_v[pl.ds(i, L)][0]
            cl = lax.bitwise_and(rr, PW - 1)
            slot = lax.bitwise_and(iss, RING - 1)
            cs = jnp.full((L,), 0, jnp.int32) + cl
            for q in range(D // L):
                dv = q * L + lanes
                col = plsc.load_gather(panel_v.at[buf], [dv, cs])
                temp_v[pl.ds(slot * D + q * L, L)] = col
            pltpu.async_copy(
                temp_v.at[pl.ds(slot * D, D)],
                out_hbm.at[pl.ds(bb * D, D)], sem_o)
            return iss + 1

        return lax.fori_loop(0, cp, per_id, issued)

    def sweep(s, issued):
        buf = lax.bitwise_and(s, 1)
        p = s * NW + w
        _drain_panel(utabT_hbm, panel_u, p, sem_p)
        _drain_panel(rtabT_hbm, panel_r, p, sem_p)
        p_next = (s + 1) * NW + w

        @pl.when(p_next <= NPAN)
        def _():
            nbuf = lax.bitwise_and(s + 1, 1)
            _fire_panel(utabT_hbm, panel_u, nbuf, p_next, sem_p)
            _fire_panel(rtabT_hbm, panel_r, nbuf, p_next, sem_p)

        issued = extract(p, panel_u, buf, mr_u, mb_u, cnt_u, ug_hbm, issued)
        issued = extract(p, panel_r, buf, mr_r, mb_r, cnt_r, rg_hbm, issued)
        return issued

    total = lax.fori_loop(0, SLOTS, sweep, 0)

    def fin(i, carry):
        pltpu.make_async_copy(
            ug_hbm.at[pl.ds(0, D)], temp_v.at[pl.ds(0, D)], sem_o).wait()
        return carry

    lax.fori_loop(0, jnp.minimum(total, RING), fin, 0)


def _cosine_body(ug_hbm, rg_hbm, out_hbm, urows_v, rrows_v, out_v):
    w = lax.axis_index("s") * NC + lax.axis_index("c")
    base = w * BPW
    pltpu.sync_copy(ug_hbm.at[pl.ds(base * D, BPW * D)], urows_v)
    pltpu.sync_copy(rg_hbm.at[pl.ds(base * D, BPW * D)], rrows_v)

    lanes = lax.iota(jnp.int32, L)
    eps2 = jnp.float32(1e-16)

    def group(g, carry):
        flat = (g * L + lanes) * D
        dot = jnp.zeros((L,), jnp.float32)
        u2 = jnp.zeros((L,), jnp.float32)
        r2 = jnp.zeros((L,), jnp.float32)
        for d in range(D):
            uc = plsc.load_gather(urows_v, [flat + d])
            rc = plsc.load_gather(rrows_v, [flat + d])
            dot = dot + uc * rc
            u2 = u2 + uc * uc
            r2 = r2 + rc * rc
        sim = (dot * _nr_rsqrt(jnp.maximum(u2, eps2))
               * _nr_rsqrt(jnp.maximum(r2, eps2)))
        out_v[pl.ds(g * L, L)] = sim
        return carry

    lax.fori_loop(0, BPW // L, group, 0)
    pltpu.sync_copy(out_v, out_hbm.at[pl.ds(base, BPW)])


def kernel(user_ids, reel_ids, user_table, reel_table):
    uids = user_ids.astype(jnp.int32)
    rids = reel_ids.astype(jnp.int32)
    mesh = plsc.VectorSubcoreMesh(core_axis_name="c", subcore_axis_name="s")
    params = pltpu.CompilerParams(
        needs_layout_passes=False, use_tc_tiling_on_sc=True)

    gather_fn = functools.partial(
        pl.kernel,
        mesh=mesh,
        compiler_params=params,
        out_type=(
            jax.ShapeDtypeStruct((BATCH * D,), jnp.float32),
            jax.ShapeDtypeStruct((BATCH * D,), jnp.float32),
        ),
        scratch_types=[
            pltpu.VMEM((BATCH,), jnp.int32),
            pltpu.VMEM((MCAP + L,), jnp.int32),
            pltpu.VMEM((MCAP + L,), jnp.int32),
            pltpu.VMEM((MCAP + L,), jnp.int32),
            pltpu.VMEM((MCAP + L,), jnp.int32),
            pltpu.VMEM((2, D, PW), jnp.float32),
            pltpu.VMEM((2, D, PW), jnp.float32),
            pltpu.VMEM((MCAP + L,), jnp.int32),
            pltpu.VMEM((MCAP + L,), jnp.int32),
            pltpu.VMEM((RING * D,), jnp.float32),
            pltpu.SemaphoreType.DMA,
            pltpu.SemaphoreType.DMA,
        ],
    )(_gather_body)
    ug, rg = gather_fn(uids, rids, user_table.T, reel_table.T)

    cos_fn = functools.partial(
        pl.kernel,
        mesh=mesh,
        compiler_params=params,
        out_type=jax.ShapeDtypeStruct((BATCH,), jnp.float32),
        scratch_types=[
            pltpu.VMEM((BPW * D,), jnp.float32),
            pltpu.VMEM((BPW * D,), jnp.float32),
            pltpu.VMEM((BPW,), jnp.float32),
        ],
    )(_cosine_body)
    return cos_fn(ug, rg)


# speedup vs baseline: 2.7084x; 1.2545x over previous
import functools

import jax
import jax.numpy as jnp
from jax import lax
from jax.experimental import pallas as pl
from jax.experimental.pallas import tpu as pltpu
from jax.experimental.pallas import tpu_sc as plsc

BATCH = 16384
D = 64
L = 16
NC = 2
NS = 16
NW = NC * NS
BPW = BATCH // NW
NROWS = 1000001
PW = 256
PSHIFT = 8
NPAN = NROWS // PW
SLOTS = 123
MCAP = 1040
SCAN_G = BATCH // L
RING = 32


def _nr_rsqrt(x):
    i = lax.bitcast_convert_type(x, jnp.int32)
    i = jnp.int32(0x5F3759DF) - lax.shift_right_logical(i, 1)
    y = lax.bitcast_convert_type(i, jnp.float32)
    for _ in range(3):
        y = y * (jnp.float32(1.5) - jnp.float32(0.5) * x * y * y)
    return y


def _fire_panel(tab_hbm, panel_v, buf, p, sem):
    @pl.when(p < NPAN)
    def _():
        for br in range(8):
            pltpu.async_copy(
                tab_hbm.at[pl.ds(br * 8, 8), pl.ds(p * PW, PW)],
                panel_v.at[buf, pl.ds(br * 8, 8), pl.ds(0, PW)], sem)

    @pl.when(p == NPAN)
    def _():
        for d in range(D):
            pltpu.async_copy(
                tab_hbm.at[pl.ds(d, 1), pl.ds(NPAN * PW, 64)],
                panel_v.at[buf, pl.ds(d, 1), pl.ds(0, 64)], sem)


def _drain_panel(tab_hbm, panel_v, p, sem):
    @pl.when(p < NPAN)
    def _():
        for _i in range(8):
            pltpu.make_async_copy(
                tab_hbm.at[pl.ds(0, 8), pl.ds(0, PW)],
                panel_v.at[0, pl.ds(0, 8), pl.ds(0, PW)], sem).wait()

    @pl.when(p == NPAN)
    def _():
        for _i in range(D):
            pltpu.make_async_copy(
                tab_hbm.at[pl.ds(0, 1), pl.ds(0, 64)],
                panel_v.at[0, pl.ds(0, 1), pl.ds(0, 64)], sem).wait()


def _gather_body(uids_hbm, rids_hbm, utabT_hbm, rtabT_hbm, ug_hbm, rg_hbm,
                 aids_v, mr_u, mb_u, mr_r, mb_r, panel_u, panel_r,
                 pr_v, pb_v, temp_v, sem_p, sem_o):
    w = lax.axis_index("s") * NC + lax.axis_index("c")
    lanes = lax.iota(jnp.int32, L)

    for s0 in range(2):
        _fire_panel(utabT_hbm, panel_u, s0, s0 * NW + w, sem_p)
        _fire_panel(rtabT_hbm, panel_r, s0, s0 * NW + w, sem_p)

    def scan_ids(ids_hbm, mr_v, mb_v):
        pltpu.sync_copy(ids_hbm, aids_v)

        def scan(j, cnt):
            v = aids_v[pl.ds(j * L, L)]
            m = lax.eq(
                lax.bitwise_and(lax.shift_right_logical(v, PSHIFT), 31), w)
            plsc.store_compressed(mr_v.at[pl.ds(cnt, L)], v, mask=m)
            plsc.store_compressed(
                mb_v.at[pl.ds(cnt, L)], j * L + lanes, mask=m)
            return cnt + plsc.all_reduce_population_count(m)[0]

        return lax.fori_loop(0, SCAN_G, scan, 0)

    cnt_u = scan_ids(uids_hbm, mr_u, mb_u)
    cnt_r = scan_ids(rids_hbm, mr_r, mb_r)

    def extract(p, panel_v, buf, mr_v, mb_v, cnt, out_hbm, issued):
        def pscan(j, c2):
            valid = (j * L + lanes) < cnt
            rv = mr_v[pl.ds(j * L, L)]
            bv = mb_v[pl.ds(j * L, L)]
            m = valid & lax.eq(lax.shift_right_logical(rv, PSHIFT), p)
            plsc.store_compressed(pr_v.at[pl.ds(c2, L)], rv, mask=m)
            plsc.store_compressed(pb_v.at[pl.ds(c2, L)], bv, mask=m)
            return c2 + plsc.all_reduce_population_count(m)[0]

        cp = lax.fori_loop(0, lax.shift_right_logical(cnt + (L - 1), 4),
                           pscan, 0)

        def per_id(i, iss):
            @pl.when(iss >= RING)
            def _():
                pltpu.make_async_copy(
                    ug_hbm.at[pl.ds(0, D)],
                    temp_v.at[pl.ds(0, D)], sem_o).wait()

            rr = pr_v[pl.ds(i, L)][0]
            bb = pb_v[pl.ds(i, L)][0]
            cl = lax.bitwise_and(rr, PW - 1)
            slot = lax.bitwise_and(iss, RING - 1)
            cs = jnp.full((L,), 0, jnp.int32) + cl
            for q in range(D // L):
                dv = q * L + lanes
                col = plsc.load_gather(panel_v.at[buf], [dv, cs])
                temp_v[pl.ds(slot * D + q * L, L)] = col
            pltpu.async_copy(
                temp_v.at[pl.ds(slot * D, D)],
                out_hbm.at[pl.ds(bb * D, D)], sem_o)
            return iss + 1

        return lax.fori_loop(0, cp, per_id, issued)

    def sweep(s, issued):
        buf = lax.rem(s, 3)
        p = s * NW + w
        _drain_panel(utabT_hbm, panel_u, p, sem_p)
        _drain_panel(rtabT_hbm, panel_r, p, sem_p)
        p_next = (s + 2) * NW + w

        @pl.when(p_next <= NPAN)
        def _():
            nbuf = lax.rem(s + 2, 3)
            _fire_panel(utabT_hbm, panel_u, nbuf, p_next, sem_p)
            _fire_panel(rtabT_hbm, panel_r, nbuf, p_next, sem_p)

        issued = extract(p, panel_u, buf, mr_u, mb_u, cnt_u, ug_hbm, issued)
        issued = extract(p, panel_r, buf, mr_r, mb_r, cnt_r, rg_hbm, issued)
        return issued

    total = lax.fori_loop(0, SLOTS, sweep, 0)

    def fin(i, carry):
        pltpu.make_async_copy(
            ug_hbm.at[pl.ds(0, D)], temp_v.at[pl.ds(0, D)], sem_o).wait()
        return carry

    lax.fori_loop(0, jnp.minimum(total, RING), fin, 0)


def _cosine_body(ug_hbm, rg_hbm, out_hbm, urows_v, rrows_v, out_v):
    w = lax.axis_index("s") * NC + lax.axis_index("c")
    base = w * BPW
    pltpu.sync_copy(ug_hbm.at[pl.ds(base * D, BPW * D)], urows_v)
    pltpu.sync_copy(rg_hbm.at[pl.ds(base * D, BPW * D)], rrows_v)

    lanes = lax.iota(jnp.int32, L)
    eps2 = jnp.float32(1e-16)

    def group(g, carry):
        flat = (g * L + lanes) * D
        dot = jnp.zeros((L,), jnp.float32)
        u2 = jnp.zeros((L,), jnp.float32)
        r2 = jnp.zeros((L,), jnp.float32)
        for d in range(D):
            uc = plsc.load_gather(urows_v, [flat + d])
            rc = plsc.load_gather(rrows_v, [flat + d])
            dot = dot + uc * rc
            u2 = u2 + uc * uc
            r2 = r2 + rc * rc
        sim = (dot * _nr_rsqrt(jnp.maximum(u2, eps2))
               * _nr_rsqrt(jnp.maximum(r2, eps2)))
        out_v[pl.ds(g * L, L)] = sim
        return carry

    lax.fori_loop(0, BPW // L, group, 0)
    pltpu.sync_copy(out_v, out_hbm.at[pl.ds(base, BPW)])


def kernel(user_ids, reel_ids, user_table, reel_table):
    uids = user_ids.astype(jnp.int32)
    rids = reel_ids.astype(jnp.int32)
    mesh = plsc.VectorSubcoreMesh(core_axis_name="c", subcore_axis_name="s")
    params = pltpu.CompilerParams(
        needs_layout_passes=False, use_tc_tiling_on_sc=True)

    gather_fn = functools.partial(
        pl.kernel,
        mesh=mesh,
        compiler_params=params,
        out_type=(
            jax.ShapeDtypeStruct((BATCH * D,), jnp.float32),
            jax.ShapeDtypeStruct((BATCH * D,), jnp.float32),
        ),
        scratch_types=[
            pltpu.VMEM((BATCH,), jnp.int32),
            pltpu.VMEM((MCAP + L,), jnp.int32),
            pltpu.VMEM((MCAP + L,), jnp.int32),
            pltpu.VMEM((MCAP + L,), jnp.int32),
            pltpu.VMEM((MCAP + L,), jnp.int32),
            pltpu.VMEM((3, D, PW), jnp.float32),
            pltpu.VMEM((3, D, PW), jnp.float32),
            pltpu.VMEM((MCAP + L,), jnp.int32),
            pltpu.VMEM((MCAP + L,), jnp.int32),
            pltpu.VMEM((RING * D,), jnp.float32),
            pltpu.SemaphoreType.DMA,
            pltpu.SemaphoreType.DMA,
        ],
    )(_gather_body)
    ug, rg = gather_fn(uids, rids, user_table.T, reel_table.T)

    cos_fn = functools.partial(
        pl.kernel,
        mesh=mesh,
        compiler_params=params,
        out_type=jax.ShapeDtypeStruct((BATCH,), jnp.float32),
        scratch_types=[
            pltpu.VMEM((BPW * D,), jnp.float32),
            pltpu.VMEM((BPW * D,), jnp.float32),
            pltpu.VMEM((BPW,), jnp.float32),
        ],
    )(_cosine_body)
    return cos_fn(ug, rg)


# speedup vs baseline: 2.7505x; 1.0156x over previous
import functools

import jax
import jax.numpy as jnp
from jax import lax
from jax.experimental import pallas as pl
from jax.experimental.pallas import tpu as pltpu
from jax.experimental.pallas import tpu_sc as plsc

BATCH = 16384
D = 64
L = 16
NC = 2
NS = 16
NW = NC * NS
BPW = BATCH // NW
NROWS = 1000001
PW = 256
PSHIFT = 8
NPAN = NROWS // PW
SLOTS = 123
MCAP = 1040
SCAN_G = BATCH // L
RING = 32


def _nr_rsqrt(x):
    i = lax.bitcast_convert_type(x, jnp.int32)
    i = jnp.int32(0x5F3759DF) - lax.shift_right_logical(i, 1)
    y = lax.bitcast_convert_type(i, jnp.float32)
    for _ in range(3):
        y = y * (jnp.float32(1.5) - jnp.float32(0.5) * x * y * y)
    return y


def _fire_panel(tab_hbm, panel_v, buf, p, sem):
    @pl.when(p < NPAN)
    def _():
        for br in range(8):
            pltpu.async_copy(
                tab_hbm.at[pl.ds(br * 8, 8), pl.ds(p * PW, PW)],
                panel_v.at[buf, pl.ds(br * 8, 8), pl.ds(0, PW)], sem)

    @pl.when(p == NPAN)
    def _():
        for d in range(D):
            pltpu.async_copy(
                tab_hbm.at[pl.ds(d, 1), pl.ds(NPAN * PW, 64)],
                panel_v.at[buf, pl.ds(d, 1), pl.ds(0, 64)], sem)


def _drain_panel(tab_hbm, panel_v, p, sem):
    @pl.when(p < NPAN)
    def _():
        for _i in range(8):
            pltpu.make_async_copy(
                tab_hbm.at[pl.ds(0, 8), pl.ds(0, PW)],
                panel_v.at[0, pl.ds(0, 8), pl.ds(0, PW)], sem).wait()

    @pl.when(p == NPAN)
    def _():
        for _i in range(D):
            pltpu.make_async_copy(
                tab_hbm.at[pl.ds(0, 1), pl.ds(0, 64)],
                panel_v.at[0, pl.ds(0, 1), pl.ds(0, 64)], sem).wait()


def _gather_body(uids_hbm, rids_hbm, utabT_hbm, rtabT_hbm, ug_hbm, rg_hbm,
                 aids_v, mr_u, mb_u, mr_r, mb_r, panel_u, panel_r,
                 pr_v, pb_v, temp_v, sem_p, sem_o):
    w = lax.axis_index("s") * NC + lax.axis_index("c")
    lanes = lax.iota(jnp.int32, L)

    for s0 in range(2):
        _fire_panel(utabT_hbm, panel_u, s0, s0 * NW + w, sem_p)
        _fire_panel(rtabT_hbm, panel_r, s0, s0 * NW + w, sem_p)

    def scan_ids(ids_hbm, mr_v, mb_v):
        pltpu.sync_copy(ids_hbm, aids_v)

        def scan(j, cnt):
            v = aids_v[pl.ds(j * L, L)]
            m = lax.eq(
                lax.bitwise_and(lax.shift_right_logical(v, PSHIFT), 31), w)
            plsc.store_compressed(mr_v.at[pl.ds(cnt, L)], v, mask=m)
            plsc.store_compressed(
                mb_v.at[pl.ds(cnt, L)], j * L + lanes, mask=m)
            return cnt + plsc.all_reduce_population_count(m)[0]

        return lax.fori_loop(0, SCAN_G, scan, 0)

    cnt_u = scan_ids(uids_hbm, mr_u, mb_u)
    cnt_r = scan_ids(rids_hbm, mr_r, mb_r)

    def extract(p, panel_v, buf, mr_v, mb_v, cnt, out_hbm, issued):
        def pscan(j, c2):
            valid = (j * L + lanes) < cnt
            rv = mr_v[pl.ds(j * L, L)]
            bv = mb_v[pl.ds(j * L, L)]
            m = valid & lax.eq(lax.shift_right_logical(rv, PSHIFT), p)
            plsc.store_compressed(pr_v.at[pl.ds(c2, L)], rv, mask=m)
            plsc.store_compressed(pb_v.at[pl.ds(c2, L)], bv, mask=m)
            return c2 + plsc.all_reduce_population_count(m)[0]

        cp = lax.fori_loop(0, lax.shift_right_logical(cnt + (L - 1), 4),
                           pscan, 0)

        def per_id(i, iss):
            @pl.when(iss >= RING)
            def _():
                pltpu.make_async_copy(
                    ug_hbm.at[pl.ds(0, D)],
                    temp_v.at[pl.ds(0, D)], sem_o).wait()

            rr = pr_v[pl.ds(i, L)][0]
            bb = pb_v[pl.ds(i, L)][0]
            cl = lax.bitwise_and(rr, PW - 1)
            slot = lax.bitwise_and(iss, RING - 1)
            cs = jnp.full((L,), 0, jnp.int32) + cl
            for q in range(D // L):
                dv = q * L + lanes
                col = plsc.load_gather(panel_v.at[buf], [dv, cs])
                temp_v[pl.ds(slot * D + q * L, L)] = col
            pltpu.async_copy(
                temp_v.at[pl.ds(slot * D, D)],
                out_hbm.at[pl.ds(bb * D, D)], sem_o)
            return iss + 1

        return lax.fori_loop(0, cp, per_id, issued)

    def sweep(s, issued):
        buf = lax.rem(s, 3)
        p = s * NW + w
        _drain_panel(utabT_hbm, panel_u, p, sem_p)
        _drain_panel(rtabT_hbm, panel_r, p, sem_p)
        p_next = (s + 2) * NW + w

        @pl.when(p_next <= NPAN)
        def _():
            nbuf = lax.rem(s + 2, 3)
            _fire_panel(utabT_hbm, panel_u, nbuf, p_next, sem_p)
            _fire_panel(rtabT_hbm, panel_r, nbuf, p_next, sem_p)

        issued = extract(p, panel_u, buf, mr_u, mb_u, cnt_u, ug_hbm, issued)
        issued = extract(p, panel_r, buf, mr_r, mb_r, cnt_r, rg_hbm, issued)
        return issued

    total = lax.fori_loop(0, SLOTS, sweep, 0)

    def fin(i, carry):
        pltpu.make_async_copy(
            ug_hbm.at[pl.ds(0, D)], temp_v.at[pl.ds(0, D)], sem_o).wait()
        return carry

    lax.fori_loop(0, jnp.minimum(total, RING), fin, 0)


CB = 64
NCB = BPW // CB


def _cosine_body(ug_hbm, rg_hbm, out_hbm, urows_v, rrows_v, out_v, sem):
    w = lax.axis_index("s") * NC + lax.axis_index("c")
    base = w * BPW

    def fire_block(k, buf):
        off = (base + k * CB) * D
        pltpu.async_copy(ug_hbm.at[pl.ds(off, CB * D)],
                         urows_v.at[pl.ds(buf * CB * D, CB * D)], sem)
        pltpu.async_copy(rg_hbm.at[pl.ds(off, CB * D)],
                         rrows_v.at[pl.ds(buf * CB * D, CB * D)], sem)

    def drain_block():
        for _i in range(2):
            pltpu.make_async_copy(
                ug_hbm.at[pl.ds(0, CB * D)],
                urows_v.at[pl.ds(0, CB * D)], sem).wait()

    fire_block(0, 0)
    lanes = lax.iota(jnp.int32, L)
    eps2 = jnp.float32(1e-16)

    def block(k, carry):
        buf = lax.bitwise_and(k, 1)
        drain_block()

        @pl.when(k + 1 < NCB)
        def _():
            fire_block(k + 1, lax.bitwise_and(k + 1, 1))

        def group(g, carry2):
            flat = buf * (CB * D) + (g * L + lanes) * D
            dot = jnp.zeros((L,), jnp.float32)
            u2 = jnp.zeros((L,), jnp.float32)
            r2 = jnp.zeros((L,), jnp.float32)
            for d in range(D):
                uc = plsc.load_gather(urows_v, [flat + d])
                rc = plsc.load_gather(rrows_v, [flat + d])
                dot = dot + uc * rc
                u2 = u2 + uc * uc
                r2 = r2 + rc * rc
            sim = (dot * _nr_rsqrt(jnp.maximum(u2, eps2))
                   * _nr_rsqrt(jnp.maximum(r2, eps2)))
            out_v[pl.ds(k * CB + g * L, L)] = sim
            return carry2

        lax.fori_loop(0, CB // L, group, 0)
        return carry

    lax.fori_loop(0, NCB, block, 0)
    pltpu.sync_copy(out_v, out_hbm.at[pl.ds(base, BPW)])


def kernel(user_ids, reel_ids, user_table, reel_table):
    uids = user_ids.astype(jnp.int32)
    rids = reel_ids.astype(jnp.int32)
    mesh = plsc.VectorSubcoreMesh(core_axis_name="c", subcore_axis_name="s")
    params = pltpu.CompilerParams(
        needs_layout_passes=False, use_tc_tiling_on_sc=True)

    gather_fn = functools.partial(
        pl.kernel,
        mesh=mesh,
        compiler_params=params,
        out_type=(
            jax.ShapeDtypeStruct((BATCH * D,), jnp.float32),
            jax.ShapeDtypeStruct((BATCH * D,), jnp.float32),
        ),
        scratch_types=[
            pltpu.VMEM((BATCH,), jnp.int32),
            pltpu.VMEM((MCAP + L,), jnp.int32),
            pltpu.VMEM((MCAP + L,), jnp.int32),
            pltpu.VMEM((MCAP + L,), jnp.int32),
            pltpu.VMEM((MCAP + L,), jnp.int32),
            pltpu.VMEM((3, D, PW), jnp.float32),
            pltpu.VMEM((3, D, PW), jnp.float32),
            pltpu.VMEM((MCAP + L,), jnp.int32),
            pltpu.VMEM((MCAP + L,), jnp.int32),
            pltpu.VMEM((RING * D,), jnp.float32),
            pltpu.SemaphoreType.DMA,
            pltpu.SemaphoreType.DMA,
        ],
    )(_gather_body)
    ug, rg = gather_fn(uids, rids, user_table.T, reel_table.T)

    cos_fn = functools.partial(
        pl.kernel,
        mesh=mesh,
        compiler_params=params,
        out_type=jax.ShapeDtypeStruct((BATCH,), jnp.float32),
        scratch_types=[
            pltpu.VMEM((2 * CB * D,), jnp.float32),
            pltpu.VMEM((2 * CB * D,), jnp.float32),
            pltpu.VMEM((BPW,), jnp.float32),
            pltpu.SemaphoreType.DMA,
        ],
    )(_cosine_body)
    return cos_fn(ug, rg)


# speedup vs baseline: 2.9149x; 1.0598x over previous
import functools

import jax
import jax.numpy as jnp
from jax import lax
from jax.experimental import pallas as pl
from jax.experimental.pallas import tpu as pltpu
from jax.experimental.pallas import tpu_sc as plsc

BATCH = 16384
D = 64
L = 16
NC = 2
NS = 16
NW = NC * NS
BPW = BATCH // NW
NROWS = 1000001
PW = 512
PSHIFT = 9
NPAN = NROWS // PW
SLOTS = 123
MCAP = 1552
SCHUNK = 2048
RING = 32


def _nr_rsqrt(x):
    i = lax.bitcast_convert_type(x, jnp.int32)
    i = jnp.int32(0x5F3759DF) - lax.shift_right_logical(i, 1)
    y = lax.bitcast_convert_type(i, jnp.float32)
    for _ in range(3):
        y = y * (jnp.float32(1.5) - jnp.float32(0.5) * x * y * y)
    return y


def _fire_panel(tab_hbm, panel_v, buf, p, sem):
    @pl.when(p < NPAN)
    def _():
        for br in range(8):
            pltpu.async_copy(
                tab_hbm.at[pl.ds(br * 8, 8), pl.ds(p * PW, PW)],
                panel_v.at[buf, pl.ds(br * 8, 8), pl.ds(0, PW)], sem)

    @pl.when(p == NPAN)
    def _():
        for d in range(D):
            pltpu.async_copy(
                tab_hbm.at[pl.ds(d, 1), pl.ds(NPAN * PW, 64)],
                panel_v.at[buf, pl.ds(d, 1), pl.ds(0, 64)], sem)


def _drain_panel(tab_hbm, panel_v, p, sem):
    @pl.when(p < NPAN)
    def _():
        for _i in range(8):
            pltpu.make_async_copy(
                tab_hbm.at[pl.ds(0, 8), pl.ds(0, PW)],
                panel_v.at[0, pl.ds(0, 8), pl.ds(0, PW)], sem).wait()

    @pl.when(p == NPAN)
    def _():
        for _i in range(D):
            pltpu.make_async_copy(
                tab_hbm.at[pl.ds(0, 1), pl.ds(0, 64)],
                panel_v.at[0, pl.ds(0, 1), pl.ds(0, 64)], sem).wait()


def _gather_body(uids_hbm, rids_hbm, utabT_hbm, rtabT_hbm, ug_hbm, rg_hbm,
                 aids_v, mr_v, mb_v, panel_v,
                 pr_v, pb_v, temp_v, sem_p, sem_o):
    c = lax.axis_index("c")
    s = lax.axis_index("s")
    lanes = lax.iota(jnp.int32, L)

    def run_table(ids_hbm, tab_hbm, out_hbm):
        for s0 in range(2):
            _fire_panel(tab_hbm, panel_v, s0, s0 * NS + s, sem_p)

        def scan_chunk(ch, cnt):
            pltpu.sync_copy(aids_hbm_slice(ids_hbm, ch), aids_v)

            def scan(j, cnt2):
                v = aids_v[pl.ds(j * L, L)]
                m = lax.eq(
                    lax.bitwise_and(
                        lax.shift_right_logical(v, PSHIFT), NS - 1), s)
                plsc.store_compressed(mr_v.at[pl.ds(cnt2, L)], v, mask=m)
                plsc.store_compressed(
                    mb_v.at[pl.ds(cnt2, L)],
                    ch * SCHUNK + j * L + lanes, mask=m)
                return cnt2 + plsc.all_reduce_population_count(m)[0]

            return lax.fori_loop(0, SCHUNK // L, scan, cnt)

        def aids_hbm_slice(ids_hbm_, ch):
            return ids_hbm_.at[pl.ds(ch * SCHUNK, SCHUNK)]

        cnt = lax.fori_loop(0, BATCH // SCHUNK, scan_chunk, 0)

        def extract(p, buf, issued):
            def pscan(j, c2):
                valid = (j * L + lanes) < cnt
                rv = mr_v[pl.ds(j * L, L)]
                bv = mb_v[pl.ds(j * L, L)]
                m = valid & lax.eq(lax.shift_right_logical(rv, PSHIFT), p)
                plsc.store_compressed(pr_v.at[pl.ds(c2, L)], rv, mask=m)
                plsc.store_compressed(pb_v.at[pl.ds(c2, L)], bv, mask=m)
                return c2 + plsc.all_reduce_population_count(m)[0]

            cp = lax.fori_loop(0, lax.shift_right_logical(cnt + (L - 1), 4),
                               pscan, 0)

            def per_id(i, iss):
                @pl.when(iss >= RING)
                def _():
                    pltpu.make_async_copy(
                        ug_hbm.at[pl.ds(0, D)],
                        temp_v.at[pl.ds(0, D)], sem_o).wait()

                rr = pr_v[pl.ds(i, L)][0]
                bb = pb_v[pl.ds(i, L)][0]
                cl = lax.bitwise_and(rr, PW - 1)
                slot = lax.bitwise_and(iss, RING - 1)
                cs = jnp.full((L,), 0, jnp.int32) + cl
                for q in range(D // L):
                    dv = q * L + lanes
                    col = plsc.load_gather(panel_v.at[buf], [dv, cs])
                    temp_v[pl.ds(slot * D + q * L, L)] = col
                pltpu.async_copy(
                    temp_v.at[pl.ds(slot * D, D)],
                    out_hbm.at[pl.ds(bb * D, D)], sem_o)
                return iss + 1

            return lax.fori_loop(0, cp, per_id, issued)

        def sweep(sl, issued):
            buf = lax.rem(sl, 3)
            p = sl * NS + s
            _drain_panel(tab_hbm, panel_v, p, sem_p)
            p_next = (sl + 2) * NS + s

            @pl.when(p_next <= NPAN)
            def _():
                _fire_panel(tab_hbm, panel_v, lax.rem(sl + 2, 3), p_next,
                            sem_p)

            return extract(p, buf, issued)

        total = lax.fori_loop(0, SLOTS, sweep, 0)

        def fin(i, carry):
            pltpu.make_async_copy(
                ug_hbm.at[pl.ds(0, D)], temp_v.at[pl.ds(0, D)], sem_o).wait()
            return carry

        lax.fori_loop(0, jnp.minimum(total, RING), fin, 0)

    @pl.when(c == 0)
    def _():
        run_table(uids_hbm, utabT_hbm, ug_hbm)

    @pl.when(c == 1)
    def _():
        run_table(rids_hbm, rtabT_hbm, rg_hbm)


CB = 64
NCB = BPW // CB


def _cosine_body(ug_hbm, rg_hbm, out_hbm, urows_v, rrows_v, out_v, sem):
    w = lax.axis_index("s") * NC + lax.axis_index("c")
    base = w * BPW

    def fire_block(k, buf):
        off = (base + k * CB) * D
        pltpu.async_copy(ug_hbm.at[pl.ds(off, CB * D)],
                         urows_v.at[pl.ds(buf * CB * D, CB * D)], sem)
        pltpu.async_copy(rg_hbm.at[pl.ds(off, CB * D)],
                         rrows_v.at[pl.ds(buf * CB * D, CB * D)], sem)

    def drain_block():
        for _i in range(2):
            pltpu.make_async_copy(
                ug_hbm.at[pl.ds(0, CB * D)],
                urows_v.at[pl.ds(0, CB * D)], sem).wait()

    fire_block(0, 0)
    lanes = lax.iota(jnp.int32, L)
    eps2 = jnp.float32(1e-16)

    def block(k, carry):
        buf = lax.bitwise_and(k, 1)
        drain_block()

        @pl.when(k + 1 < NCB)
        def _():
            fire_block(k + 1, lax.bitwise_and(k + 1, 1))

        def group(g, carry2):
            flat = buf * (CB * D) + (g * L + lanes) * D
            dot = jnp.zeros((L,), jnp.float32)
            u2 = jnp.zeros((L,), jnp.float32)
            r2 = jnp.zeros((L,), jnp.float32)
            for d in range(D):
                uc = plsc.load_gather(urows_v, [flat + d])
                rc = plsc.load_gather(rrows_v, [flat + d])
                dot = dot + uc * rc
                u2 = u2 + uc * uc
                r2 = r2 + rc * rc
            sim = (dot * _nr_rsqrt(jnp.maximum(u2, eps2))
                   * _nr_rsqrt(jnp.maximum(r2, eps2)))
            out_v[pl.ds(k * CB + g * L, L)] = sim
            return carry2

        lax.fori_loop(0, CB // L, group, 0)
        return carry

    lax.fori_loop(0, NCB, block, 0)
    pltpu.sync_copy(out_v, out_hbm.at[pl.ds(base, BPW)])


def kernel(user_ids, reel_ids, user_table, reel_table):
    uids = user_ids.astype(jnp.int32)
    rids = reel_ids.astype(jnp.int32)
    mesh = plsc.VectorSubcoreMesh(core_axis_name="c", subcore_axis_name="s")
    params = pltpu.CompilerParams(
        needs_layout_passes=False, use_tc_tiling_on_sc=True)

    gather_fn = functools.partial(
        pl.kernel,
        mesh=mesh,
        compiler_params=params,
        out_type=(
            jax.ShapeDtypeStruct((BATCH * D,), jnp.float32),
            jax.ShapeDtypeStruct((BATCH * D,), jnp.float32),
        ),
        scratch_types=[
            pltpu.VMEM((SCHUNK,), jnp.int32),
            pltpu.VMEM((MCAP + L,), jnp.int32),
            pltpu.VMEM((MCAP + L,), jnp.int32),
            pltpu.VMEM((3, D, PW), jnp.float32),
            pltpu.VMEM((MCAP + L,), jnp.int32),
            pltpu.VMEM((MCAP + L,), jnp.int32),
            pltpu.VMEM((RING * D,), jnp.float32),
            pltpu.SemaphoreType.DMA,
            pltpu.SemaphoreType.DMA,
        ],
    )(_gather_body)
    ug, rg = gather_fn(uids, rids, user_table.T, reel_table.T)

    cos_fn = functools.partial(
        pl.kernel,
        mesh=mesh,
        compiler_params=params,
        out_type=jax.ShapeDtypeStruct((BATCH,), jnp.float32),
        scratch_types=[
            pltpu.VMEM((2 * CB * D,), jnp.float32),
            pltpu.VMEM((2 * CB * D,), jnp.float32),
            pltpu.VMEM((BPW,), jnp.float32),
            pltpu.SemaphoreType.DMA,
        ],
    )(_cosine_body)
    return cos_fn(ug, rg)
